# baseline (speedup 1.0000x reference)
import functools

import jax
import jax.numpy as jnp
from jax import lax
from jax.experimental import pallas as pl
from jax.experimental.pallas import tpu as pltpu
from jax.experimental.pallas import tpu_sc as plsc

NC = 2
NS = 16
L = 16
NW = NC * NS

CH = 512


def _sc_gather(x_t, tgt_flat, E, F):
    CG = 400
    per_w = E // NW
    nchunk = per_w // CG
    mesh = plsc.VectorSubcoreMesh(core_axis_name="c", subcore_axis_name="s")

    @functools.partial(
        pl.kernel,
        out_type=jax.ShapeDtypeStruct((E, F), jnp.float32),
        mesh=mesh,
        scratch_types=[
            pltpu.VMEM((per_w,), jnp.int32),
            pltpu.VMEM((2, CG, F), jnp.float32),
            pltpu.SemaphoreType.DMA,
            pltpu.SemaphoreType.DMA,
            pltpu.SemaphoreType.DMA,
            pltpu.SemaphoreType.DMA,
        ],
    )
    def k(x_t_hbm, tgt_hbm, out_hbm, idx_v, rows_v, g0, g1, w0, w1):
        gsems = (g0, g1)
        wsems = (w0, w1)
        cid = lax.axis_index("c")
        sid = lax.axis_index("s")
        wid = sid * NC + cid
        e0 = wid * per_w

        pltpu.sync_copy(tgt_hbm.at[pl.ds(e0, per_w)], idx_v)
        pltpu.async_copy(x_t_hbm.at[idx_v.at[pl.ds(0, CG)]], rows_v.at[0],
                         gsems[0])

        def chunk_body(c, carry):
            for slot in range(2):
                @pl.when(c * 2 + slot < nchunk)
                def _():
                    cc = c * 2 + slot
                    nxt = 1 - slot

                    @pl.when(cc + 1 < nchunk)
                    def _():
                        @pl.when(cc >= 1)
                        def _():
                            pltpu.make_async_copy(
                                rows_v.at[nxt],
                                out_hbm.at[pl.ds(e0, CG)],
                                wsems[nxt]).wait()
                        pltpu.async_copy(
                            x_t_hbm.at[idx_v.at[pl.ds((cc + 1) * CG, CG)]],
                            rows_v.at[nxt], gsems[nxt])

                    pltpu.make_async_copy(
                        x_t_hbm.at[idx_v.at[pl.ds(0, CG)]],
                        rows_v.at[slot], gsems[slot]).wait()
                    pltpu.async_copy(
                        rows_v.at[slot],
                        out_hbm.at[pl.ds(e0 + cc * CG, CG)], wsems[slot])

            return carry

        lax.fori_loop(0, (nchunk + 1) // 2, chunk_body, 0)
        for slot in range(2):
            @pl.when(jnp.logical_and(nchunk > slot, True))
            def _():
                pltpu.make_async_copy(
                    rows_v.at[slot], out_hbm.at[pl.ds(e0, CG)],
                    wsems[slot]).wait()

    return k(x_t, tgt_flat)


def _tc_mlp1(gathered, edge_attr, W1a, b1a, W1b, b1b, E, F, FM):
    B = 2000
    grid = (E // B,)

    def body(g_ref, e_ref, wa_ref, ba_ref, wb_ref, bb_ref, o_ref):
        x = jnp.concatenate([g_ref[...], e_ref[...]], axis=1)
        h = lax.dot_general(x, wa_ref[...], (((1,), (1,)), ((), ())),
                            preferred_element_type=jnp.float32) + ba_ref[...]
        h = jnp.where(h >= 0, h, 0.1 * h)
        mm = lax.dot_general(h, wb_ref[...], (((1,), (1,)), ((), ())),
                             preferred_element_type=jnp.float32) + bb_ref[...]
        o_ref[0] = mm[:, :F]
        o_ref[1] = mm[:, F:]

    return pl.pallas_call(
        body,
        grid=grid,
        in_specs=[
            pl.BlockSpec((B, F), lambda i: (i, 0)),
            pl.BlockSpec((B, F), lambda i: (i, 0)),
            pl.BlockSpec((FM, FM), lambda i: (0, 0)),
            pl.BlockSpec((1, FM), lambda i: (0, 0)),
            pl.BlockSpec((FM, FM), lambda i: (0, 0)),
            pl.BlockSpec((1, FM), lambda i: (0, 0)),
        ],
        out_specs=pl.BlockSpec((2, B, F), lambda i: (0, i, 0)),
        out_shape=jax.ShapeDtypeStruct((2, E, F), jnp.float32),
    )(gathered, edge_attr, W1a, b1a.reshape(1, FM), W1b, b1b.reshape(1, FM))


def _sc_moments(src2d, msg, E, SP, F, FM):
    CM = 128
    FQ = 64
    NB = 3
    nchunk = E // CM
    ngroup = (nchunk + 7) // 8
    gpt = (ngroup + NS - 1) // NS
    cpt = gpt * 8
    rows_per_tile = SP // NS
    mesh = plsc.VectorSubcoreMesh(core_axis_name="c", subcore_axis_name="s")

    @functools.partial(
        pl.kernel,
        out_type=(
            jax.ShapeDtypeStruct((4, SP, FM), jnp.float32),
            jax.ShapeDtypeStruct((SP, L), jnp.float32),
        ),
        mesh=mesh,
        scratch_types=[
            pltpu.VMEM_SHARED((SP, FQ), jnp.float32),
            pltpu.VMEM_SHARED((SP, L), jnp.float32),
            pltpu.VMEM((NB, CM, L), jnp.float32),
            pltpu.VMEM((NB, CM, FQ), jnp.float32),
            pltpu.VMEM((cpt, 128), jnp.int32),
            pltpu.VMEM((CM, L), jnp.float32),
            pltpu.SemaphoreType.DMA,
            pltpu.SemaphoreType.DMA,
            pltpu.SemaphoreType.DMA,
            pltpu.SemaphoreType.DMA,
            pltpu.SemaphoreType.DMA,
            pltpu.SemaphoreType.DMA,
        ],
        compiler_params=pltpu.CompilerParams(use_tc_tiling_on_sc=False),
    )
    def k(src_hbm, msg_hbm, mom_hbm, cnt_hbm, acc_sh, cnt_sh, rbufs, wbufs,
          src_tile, ones_buf, rs0, rs1, rs2, as0, as1, as2):
        rsems = (rs0, rs1, rs2)
        asems = (as0, as1, as2)
        cid = lax.axis_index("c")
        sid = lax.axis_index("s")
        row0 = sid * rows_per_tile
        kb = sid * cpt
        nch_t = jnp.minimum(nchunk - kb, cpt)

        pltpu.sync_copy(src_hbm.at[pl.ds(kb, cpt)], src_tile)

        def fill_ones(i, c):
            ones_buf[i] = jnp.full((L,), 1.0, jnp.float32)
            return c

        lax.fori_loop(0, CM, fill_ones, 0)

        nfull = rows_per_tile // CM
        nrem = rows_per_tile % CM
        for p in range(8):
            col0 = p * L
            zbuf = wbufs.at[0]

            def zero_buf(i, c):
                for g in range(FQ // L):
                    zbuf[i, pl.ds(g * L, L)] = jnp.zeros((L,), jnp.float32)
                return c

            lax.fori_loop(0, CM, zero_buf, 0)
            for q in range(nfull):
                pltpu.sync_copy(zbuf, acc_sh.at[pl.ds(row0 + q * CM, CM)])
            if nrem:
                pltpu.sync_copy(
                    zbuf.at[pl.ds(0, nrem)],
                    acc_sh.at[pl.ds(row0 + nfull * CM, nrem)],
                )
            if p == 0:
                for q in range(nfull):
                    pltpu.sync_copy(
                        zbuf.at[pl.ds(0, CM), pl.ds(0, L)],
                        cnt_sh.at[pl.ds(row0 + q * CM, CM)])
                if nrem:
                    pltpu.sync_copy(
                        zbuf.at[pl.ds(0, nrem), pl.ds(0, L)],
                        cnt_sh.at[pl.ds(row0 + nfull * CM, nrem)])
            plsc.subcore_barrier()

            for slot in range(NB):
                @pl.when(slot < nch_t)
                def _():
                    pltpu.async_copy(
                        msg_hbm.at[cid, pl.ds((kb + slot) * CM, CM),
                                   pl.ds(col0, L)],
                        rbufs.at[slot], rsems[slot])

            def outer(j, carry):
                for slot in range(NB):
                    c = j * NB + slot

                    @pl.when(c < nch_t)
                    def _():
                        pltpu.make_async_copy(
                            msg_hbm.at[cid, pl.ds(kb * CM, CM),
                                       pl.ds(col0, L)],
                            rbufs.at[slot], rsems[slot]).wait()

                        @pl.when(c >= NB)
                        def _():
                            pltpu.make_async_copy(
                                wbufs.at[slot],
                                acc_sh.at[pl.ds(0, CM)],
                                asems[slot]).wait()

                        rb = rbufs.at[slot]
                        wb = wbufs.at[slot]

                        def pw(i, cc):
                            for r in range(4):
                                v = rb[i * 4 + r]
                                v2 = v * v
                                v3 = v2 * v
                                v4 = v2 * v2
                                wb[i * 4 + r, pl.ds(0, L)] = v
                                wb[i * 4 + r, pl.ds(L, L)] = v2
                                wb[i * 4 + r, pl.ds(2 * L, L)] = v3
                                wb[i * 4 + r, pl.ds(3 * L, L)] = v4
                            return cc

                        lax.fori_loop(0, CM // 4, pw, 0)
                        pltpu.async_copy(
                            wb, acc_sh.at[src_tile.at[c]], asems[slot],
                            add=True)
                        if p == 0:
                            @pl.when(cid == 0)
                            def _():
                                pltpu.sync_copy(
                                    ones_buf,
                                    cnt_sh.at[src_tile.at[c]],
                                    add=True)

                        @pl.when(c + NB < nch_t)
                        def _():
                            pltpu.async_copy(
                                msg_hbm.at[cid,
                                           pl.ds((kb + c + NB) * CM, CM),
                                           pl.ds(col0, L)],
                                rbufs.at[slot], rsems[slot])

                return carry

            lax.fori_loop(0, (cpt + NB - 1) // NB, outer, 0)
            for slot in range(NB):
                @pl.when(slot < nch_t)
                def _():
                    pltpu.make_async_copy(
                        wbufs.at[slot], acc_sh.at[pl.ds(0, CM)],
                        asems[slot]).wait()
            plsc.subcore_barrier()

            stage = wbufs.at[0]
            ocol = cid * F + col0
            for q in range(nfull):
                pltpu.sync_copy(acc_sh.at[pl.ds(row0 + q * CM, CM)], stage)
                for mk in range(4):
                    pltpu.sync_copy(
                        stage.at[pl.ds(0, CM), pl.ds(mk * L, L)],
                        mom_hbm.at[mk, pl.ds(row0 + q * CM, CM),
                                   pl.ds(ocol, L)],
                    )
            if nrem:
                pltpu.sync_copy(
                    acc_sh.at[pl.ds(row0 + nfull * CM, nrem)],
                    stage.at[pl.ds(0, nrem)],
                )
                for mk in range(4):
                    pltpu.sync_copy(
                        stage.at[pl.ds(0, nrem), pl.ds(mk * L, L)],
                        mom_hbm.at[mk, pl.ds(row0 + nfull * CM, nrem),
                                   pl.ds(ocol, L)],
                    )
            if p == 0:
                @pl.when(cid == 0)
                def _():
                    st16 = rbufs.at[0]
                    for q in range(nfull):
                        pltpu.sync_copy(
                            cnt_sh.at[pl.ds(row0 + q * CM, CM)], st16)
                        pltpu.sync_copy(
                            st16, cnt_hbm.at[pl.ds(row0 + q * CM, CM)])
                    if nrem:
                        st16r = rbufs.at[0].at[pl.ds(0, nrem)]
                        pltpu.sync_copy(
                            cnt_sh.at[pl.ds(row0 + nfull * CM, nrem)],
                            st16r)
                        pltpu.sync_copy(
                            st16r,
                            cnt_hbm.at[pl.ds(row0 + nfull * CM, nrem)])

    return k(src2d, msg)


def _tc_update(moms, cnt, x_s, u, W2a, b2a, W2b, b2b, S, F, FM, FM2):
    Bn = 1000
    grid = (S // Bn,)

    def body(mom_ref, cnt_ref, xs_ref, u_ref, wa_ref, ba_ref, wb_ref, bb_ref,
             o_ref):
        mm = mom_ref[...]
        c = cnt_ref[...][:, 0:1]
        cc = jnp.maximum(c, 1.0)
        m1 = mm[0] / cc
        m2 = mm[1] / cc
        m3 = mm[2] / cc
        m4 = mm[3] / cc
        mean = m1
        raw = m2 - mean * mean
        var = jnp.where(raw >= 0, raw, 0.01 * raw)
        std1 = jnp.sqrt(var + 1e-6)
        c3 = m3 - 3.0 * m2 * mean + 2.0 * mean * mean * mean
        c4 = (m4 - 4.0 * m3 * mean + 6.0 * m2 * mean * mean
              - 3.0 * mean * mean * mean * mean)
        skew = c3 / (std1 * std1 * std1)
        kurt = c4 / (std1 * std1 * std1 * std1)
        mean = jnp.nan_to_num(mean)
        var = jnp.nan_to_num(var)
        std = jnp.sqrt(var + 1e-6)
        skew = jnp.nan_to_num(skew)
        kurt = jnp.nan_to_num(kurt)
        ub = jnp.broadcast_to(u_ref[...], (Bn, F))
        h = jnp.concatenate([xs_ref[...], mean, std, skew, kurt, ub], axis=1)
        t = lax.dot_general(h, wa_ref[...], (((1,), (1,)), ((), ())),
                            preferred_element_type=jnp.float32) + ba_ref[...]
        t = jnp.where(t >= 0, t, 0.1 * t)
        o_ref[...] = lax.dot_general(t, wb_ref[...], (((1,), (1,)), ((), ())),
                                     preferred_element_type=jnp.float32) + bb_ref[...]

    return pl.pallas_call(
        body,
        grid=grid,
        in_specs=[
            pl.BlockSpec((4, Bn, FM), lambda i: (0, i, 0)),
            pl.BlockSpec((Bn, L), lambda i: (i, 0)),
            pl.BlockSpec((Bn, F), lambda i: (i, 0)),
            pl.BlockSpec((1, F), lambda i: (0, 0)),
            pl.BlockSpec((FM2, FM2), lambda i: (0, 0)),
            pl.BlockSpec((1, FM2), lambda i: (0, 0)),
            pl.BlockSpec((F, FM2), lambda i: (0, 0)),
            pl.BlockSpec((1, F), lambda i: (0, 0)),
        ],
        out_specs=pl.BlockSpec((Bn, F), lambda i: (i, 0)),
        out_shape=jax.ShapeDtypeStruct((S, F), jnp.float32),
    )(moms, cnt, x_s, u, W2a, b2a.reshape(1, FM2), W2b, b2b.reshape(1, F))


def _tc_norm(h2, gamma, beta, S, F):
    def body(h_ref, g_ref, b_ref, o_ref):
        h = h_ref[...]
        mu = jnp.mean(h, axis=0, keepdims=True)
        xc = h - mu
        v = jnp.mean(xc * xc, axis=0, keepdims=True)
        o_ref[...] = xc / jnp.sqrt(v + 1e-5) * g_ref[...] + b_ref[...]

    return pl.pallas_call(
        body,
        out_shape=jax.ShapeDtypeStruct((S, F), jnp.float32),
    )(h2, gamma.reshape(1, F), beta.reshape(1, F))


def kernel(x_s, x_t, edge_index, edge_attr, u, W1a, b1a, W1b, b1b, W2a, b2a,
           W2b, b2b, gamma, beta):
    S, F = x_s.shape
    E = edge_attr.shape[0]
    FM = W1a.shape[0]
    FM2 = W2a.shape[0]

    SP = ((S + 8 * NS - 1) // (8 * NS)) * (8 * NS)

    src2d = edge_index[0].reshape(E // 128, 128)
    nchunk = E // 128
    gpt = ((nchunk + 7) // 8 + NS - 1) // NS
    nrow_pad = NS * gpt * 8 - nchunk
    if nrow_pad:
        src2d = jnp.pad(src2d, ((0, nrow_pad), (0, 0)))
    tgt_flat = edge_index[1]

    gathered = _sc_gather(x_t, tgt_flat, E, F)
    msg = _tc_mlp1(gathered, edge_attr, W1a, b1a, W1b, b1b, E, F, FM)
    moms, cnt = _sc_moments(src2d, msg, E, SP, F, FM)
    h2 = _tc_update(moms[:, :S, :], cnt[:S], x_s, u, W2a, b2a, W2b, b2b,
                    S, F, FM, FM2)
    return _tc_norm(h2, gamma, beta, S, F)


# speedup vs baseline: 1.3258x; 1.3258x over previous
import functools

import jax
import jax.numpy as jnp
from jax import lax
from jax.experimental import pallas as pl
from jax.experimental.pallas import tpu as pltpu
from jax.experimental.pallas import tpu_sc as plsc

NC = 2
NS = 16
L = 16
NW = NC * NS

CH = 512


def _sc_gather(x_t, tgt_flat, E, F):
    CG = 400
    per_w = E // NW
    nchunk = per_w // CG
    mesh = plsc.VectorSubcoreMesh(core_axis_name="c", subcore_axis_name="s")

    @functools.partial(
        pl.kernel,
        out_type=jax.ShapeDtypeStruct((E, F), jnp.float32),
        mesh=mesh,
        scratch_types=[
            pltpu.VMEM((per_w,), jnp.int32),
            pltpu.VMEM((2, CG, F), jnp.float32),
            pltpu.SemaphoreType.DMA,
            pltpu.SemaphoreType.DMA,
            pltpu.SemaphoreType.DMA,
            pltpu.SemaphoreType.DMA,
        ],
    )
    def k(x_t_hbm, tgt_hbm, out_hbm, idx_v, rows_v, g0, g1, w0, w1):
        gsems = (g0, g1)
        wsems = (w0, w1)
        cid = lax.axis_index("c")
        sid = lax.axis_index("s")
        wid = sid * NC + cid
        e0 = wid * per_w

        pltpu.sync_copy(tgt_hbm.at[pl.ds(e0, per_w)], idx_v)
        pltpu.async_copy(x_t_hbm.at[idx_v.at[pl.ds(0, CG)]], rows_v.at[0],
                         gsems[0])

        def chunk_body(c, carry):
            for slot in range(2):
                @pl.when(c * 2 + slot < nchunk)
                def _():
                    cc = c * 2 + slot
                    nxt = 1 - slot

                    @pl.when(cc + 1 < nchunk)
                    def _():
                        @pl.when(cc >= 1)
                        def _():
                            pltpu.make_async_copy(
                                rows_v.at[nxt],
                                out_hbm.at[pl.ds(e0, CG)],
                                wsems[nxt]).wait()
                        pltpu.async_copy(
                            x_t_hbm.at[idx_v.at[pl.ds((cc + 1) * CG, CG)]],
                            rows_v.at[nxt], gsems[nxt])

                    pltpu.make_async_copy(
                        x_t_hbm.at[idx_v.at[pl.ds(0, CG)]],
                        rows_v.at[slot], gsems[slot]).wait()
                    pltpu.async_copy(
                        rows_v.at[slot],
                        out_hbm.at[pl.ds(e0 + cc * CG, CG)], wsems[slot])

            return carry

        lax.fori_loop(0, (nchunk + 1) // 2, chunk_body, 0)
        for slot in range(2):
            @pl.when(jnp.logical_and(nchunk > slot, True))
            def _():
                pltpu.make_async_copy(
                    rows_v.at[slot], out_hbm.at[pl.ds(e0, CG)],
                    wsems[slot]).wait()

    return k(x_t, tgt_flat)


def _tc_mlp1(gathered, edge_attr, W1a, b1a, W1b, b1b, E, F, FM):
    B = 2000
    grid = (E // B,)

    def body(g_ref, e_ref, wa_ref, ba_ref, wb_ref, bb_ref, o_ref):
        x = jnp.concatenate([g_ref[...], e_ref[...]], axis=1)
        h = lax.dot_general(x, wa_ref[...], (((1,), (1,)), ((), ())),
                            preferred_element_type=jnp.float32) + ba_ref[...]
        h = jnp.where(h >= 0, h, 0.1 * h)
        mm = lax.dot_general(h, wb_ref[...], (((1,), (1,)), ((), ())),
                             preferred_element_type=jnp.float32) + bb_ref[...]
        o_ref[0] = mm[:, :F]
        o_ref[1] = mm[:, F:]

    return pl.pallas_call(
        body,
        grid=grid,
        in_specs=[
            pl.BlockSpec((B, F), lambda i: (i, 0)),
            pl.BlockSpec((B, F), lambda i: (i, 0)),
            pl.BlockSpec((FM, FM), lambda i: (0, 0)),
            pl.BlockSpec((1, FM), lambda i: (0, 0)),
            pl.BlockSpec((FM, FM), lambda i: (0, 0)),
            pl.BlockSpec((1, FM), lambda i: (0, 0)),
        ],
        out_specs=pl.BlockSpec((2, B, F), lambda i: (0, i, 0)),
        out_shape=jax.ShapeDtypeStruct((2, E, F), jnp.float32),
    )(gathered, edge_attr, W1a, b1a.reshape(1, FM), W1b, b1b.reshape(1, FM))


def _sc_moments(src2d, msg, E, SP, F, FM):
    CM = 128
    FQ = 64
    NB = 3
    nchunk = E // CM
    ngroup = (nchunk + 7) // 8
    gpt = (ngroup + NS - 1) // NS
    cpt = gpt * 8
    rows_per_tile = SP // NS
    mesh = plsc.VectorSubcoreMesh(core_axis_name="c", subcore_axis_name="s")

    @functools.partial(
        pl.kernel,
        out_type=(
            jax.ShapeDtypeStruct((4, SP, FM), jnp.float32),
            jax.ShapeDtypeStruct((SP, L), jnp.float32),
        ),
        mesh=mesh,
        scratch_types=[
            pltpu.VMEM_SHARED((SP, FQ), jnp.float32),
            pltpu.VMEM_SHARED((SP, L), jnp.float32),
            pltpu.VMEM((NB, CM, FQ), jnp.float32),
            pltpu.VMEM((NB, CM, FQ), jnp.float32),
            pltpu.VMEM((cpt, 128), jnp.int32),
            pltpu.VMEM((CM, L), jnp.float32),
            pltpu.SemaphoreType.DMA,
            pltpu.SemaphoreType.DMA,
            pltpu.SemaphoreType.DMA,
            pltpu.SemaphoreType.DMA,
            pltpu.SemaphoreType.DMA,
            pltpu.SemaphoreType.DMA,
        ],
        compiler_params=pltpu.CompilerParams(use_tc_tiling_on_sc=False),
    )
    def k(src_hbm, msg_hbm, mom_hbm, cnt_hbm, acc_sh, cnt_sh, rbufs, wbufs,
          src_tile, ones_buf, rs0, rs1, rs2, as0, as1, as2):
        rsems = (rs0, rs1, rs2)
        asems = (as0, as1, as2)
        cid = lax.axis_index("c")
        sid = lax.axis_index("s")
        row0 = sid * rows_per_tile
        kb = sid * cpt
        nch_t = jnp.minimum(nchunk - kb, cpt)

        pltpu.sync_copy(src_hbm.at[pl.ds(kb, cpt)], src_tile)

        def fill_ones(i, c):
            ones_buf[i] = jnp.full((L,), 1.0, jnp.float32)
            return c

        lax.fori_loop(0, CM, fill_ones, 0)

        nfull = rows_per_tile // CM
        nrem = rows_per_tile % CM
        for m in range(1, 5):
            for half in range(2):
                col0 = half * FQ
                zbuf = wbufs.at[0]

                def zero_buf(i, c):
                    for g in range(FQ // L):
                        zbuf[i, pl.ds(g * L, L)] = jnp.zeros((L,), jnp.float32)
                    return c

                lax.fori_loop(0, CM, zero_buf, 0)
                for q in range(nfull):
                    pltpu.sync_copy(zbuf, acc_sh.at[pl.ds(row0 + q * CM, CM)])
                if nrem:
                    pltpu.sync_copy(
                        zbuf.at[pl.ds(0, nrem)],
                        acc_sh.at[pl.ds(row0 + nfull * CM, nrem)],
                    )
                if m == 1 and half == 0:
                    for q in range(nfull):
                        pltpu.sync_copy(
                            zbuf.at[pl.ds(0, CM), pl.ds(0, L)],
                            cnt_sh.at[pl.ds(row0 + q * CM, CM)])
                    if nrem:
                        pltpu.sync_copy(
                            zbuf.at[pl.ds(0, nrem), pl.ds(0, L)],
                            cnt_sh.at[pl.ds(row0 + nfull * CM, nrem)])
                plsc.subcore_barrier()

                for slot in range(NB):
                    @pl.when(slot < nch_t)
                    def _():
                        pltpu.async_copy(
                            msg_hbm.at[cid, pl.ds((kb + slot) * CM, CM),
                                       pl.ds(col0, FQ)],
                            rbufs.at[slot], rsems[slot])

                def outer(j, carry):
                    for slot in range(NB):
                        c = j * NB + slot

                        @pl.when(c < nch_t)
                        def _():
                            pltpu.make_async_copy(
                                msg_hbm.at[cid, pl.ds(kb * CM, CM),
                                           pl.ds(col0, FQ)],
                                rbufs.at[slot], rsems[slot]).wait()

                            @pl.when(c >= NB)
                            def _():
                                pltpu.make_async_copy(
                                    wbufs.at[slot],
                                    acc_sh.at[pl.ds(0, CM)],
                                    asems[slot]).wait()

                            rb = rbufs.at[slot]
                            wb = wbufs.at[slot]

                            def pw(i, cc):
                                for g in range(FQ // L):
                                    v = rb[i, pl.ds(g * L, L)]
                                    if m == 1:
                                        q = v
                                    elif m == 2:
                                        q = v * v
                                    elif m == 3:
                                        q = v * v * v
                                    else:
                                        v2 = v * v
                                        q = v2 * v2
                                    wb[i, pl.ds(g * L, L)] = q
                                return cc

                            lax.fori_loop(0, CM, pw, 0)
                            pltpu.async_copy(
                                wb, acc_sh.at[src_tile.at[c]], asems[slot],
                                add=True)
                            if m == 1 and half == 0:
                                @pl.when(cid == 0)
                                def _():
                                    pltpu.sync_copy(
                                        ones_buf,
                                        cnt_sh.at[src_tile.at[c]],
                                        add=True)

                            @pl.when(c + NB < nch_t)
                            def _():
                                pltpu.async_copy(
                                    msg_hbm.at[cid,
                                               pl.ds((kb + c + NB) * CM, CM),
                                               pl.ds(col0, FQ)],
                                    rbufs.at[slot], rsems[slot])

                    return carry

                lax.fori_loop(0, (cpt + NB - 1) // NB, outer, 0)
                for slot in range(NB):
                    @pl.when(slot < nch_t)
                    def _():
                        pltpu.make_async_copy(
                            wbufs.at[slot], acc_sh.at[pl.ds(0, CM)],
                            asems[slot]).wait()
                plsc.subcore_barrier()

                stage = rbufs.at[0]
                for q in range(nfull):
                    pltpu.sync_copy(acc_sh.at[pl.ds(row0 + q * CM, CM)], stage)
                    pltpu.sync_copy(
                        stage,
                        mom_hbm.at[m - 1, pl.ds(row0 + q * CM, CM),
                                   pl.ds(cid * F + col0, FQ)],
                    )
                if nrem:
                    pltpu.sync_copy(
                        acc_sh.at[pl.ds(row0 + nfull * CM, nrem)],
                        stage.at[pl.ds(0, nrem)],
                    )
                    pltpu.sync_copy(
                        stage.at[pl.ds(0, nrem)],
                        mom_hbm.at[m - 1, pl.ds(row0 + nfull * CM, nrem),
                                   pl.ds(cid * F + col0, FQ)],
                    )
                if m == 1 and half == 0:
                    @pl.when(cid == 0)
                    def _():
                        st16 = stage.at[pl.ds(0, CM), pl.ds(0, L)]
                        for q in range(nfull):
                            pltpu.sync_copy(
                                cnt_sh.at[pl.ds(row0 + q * CM, CM)], st16)
                            pltpu.sync_copy(
                                st16, cnt_hbm.at[pl.ds(row0 + q * CM, CM)])
                        if nrem:
                            st16r = stage.at[pl.ds(0, nrem), pl.ds(0, L)]
                            pltpu.sync_copy(
                                cnt_sh.at[pl.ds(row0 + nfull * CM, nrem)],
                                st16r)
                            pltpu.sync_copy(
                                st16r,
                                cnt_hbm.at[pl.ds(row0 + nfull * CM, nrem)])

    return k(src2d, msg)


def _tc_update(moms, cnt, x_s, u, W2a, b2a, W2b, b2b, S, F, FM, FM2):
    Bn = 1000
    grid = (S // Bn,)

    def body(mom_ref, cnt_ref, xs_ref, u_ref, wa_ref, ba_ref, wb_ref, bb_ref,
             o_ref):
        mm = mom_ref[...]
        c = cnt_ref[...][:, 0:1]
        cc = jnp.maximum(c, 1.0)
        m1 = mm[0] / cc
        m2 = mm[1] / cc
        m3 = mm[2] / cc
        m4 = mm[3] / cc
        mean = m1
        raw = m2 - mean * mean
        var = jnp.where(raw >= 0, raw, 0.01 * raw)
        std1 = jnp.sqrt(var + 1e-6)
        c3 = m3 - 3.0 * m2 * mean + 2.0 * mean * mean * mean
        c4 = (m4 - 4.0 * m3 * mean + 6.0 * m2 * mean * mean
              - 3.0 * mean * mean * mean * mean)
        skew = c3 / (std1 * std1 * std1)
        kurt = c4 / (std1 * std1 * std1 * std1)
        mean = jnp.nan_to_num(mean)
        var = jnp.nan_to_num(var)
        std = jnp.sqrt(var + 1e-6)
        skew = jnp.nan_to_num(skew)
        kurt = jnp.nan_to_num(kurt)
        ub = jnp.broadcast_to(u_ref[...], (Bn, F))
        h = jnp.concatenate([xs_ref[...], mean, std, skew, kurt, ub], axis=1)
        t = lax.dot_general(h, wa_ref[...], (((1,), (1,)), ((), ())),
                            preferred_element_type=jnp.float32) + ba_ref[...]
        t = jnp.where(t >= 0, t, 0.1 * t)
        o_ref[...] = lax.dot_general(t, wb_ref[...], (((1,), (1,)), ((), ())),
                                     preferred_element_type=jnp.float32) + bb_ref[...]

    return pl.pallas_call(
        body,
        grid=grid,
        in_specs=[
            pl.BlockSpec((4, Bn, FM), lambda i: (0, i, 0)),
            pl.BlockSpec((Bn, L), lambda i: (i, 0)),
            pl.BlockSpec((Bn, F), lambda i: (i, 0)),
            pl.BlockSpec((1, F), lambda i: (0, 0)),
            pl.BlockSpec((FM2, FM2), lambda i: (0, 0)),
            pl.BlockSpec((1, FM2), lambda i: (0, 0)),
            pl.BlockSpec((F, FM2), lambda i: (0, 0)),
            pl.BlockSpec((1, F), lambda i: (0, 0)),
        ],
        out_specs=pl.BlockSpec((Bn, F), lambda i: (i, 0)),
        out_shape=jax.ShapeDtypeStruct((S, F), jnp.float32),
    )(moms, cnt, x_s, u, W2a, b2a.reshape(1, FM2), W2b, b2b.reshape(1, F))


def _tc_norm(h2, gamma, beta, S, F):
    def body(h_ref, g_ref, b_ref, o_ref):
        h = h_ref[...]
        mu = jnp.mean(h, axis=0, keepdims=True)
        xc = h - mu
        v = jnp.mean(xc * xc, axis=0, keepdims=True)
        o_ref[...] = xc / jnp.sqrt(v + 1e-5) * g_ref[...] + b_ref[...]

    return pl.pallas_call(
        body,
        out_shape=jax.ShapeDtypeStruct((S, F), jnp.float32),
    )(h2, gamma.reshape(1, F), beta.reshape(1, F))


def kernel(x_s, x_t, edge_index, edge_attr, u, W1a, b1a, W1b, b1b, W2a, b2a,
           W2b, b2b, gamma, beta):
    S, F = x_s.shape
    E = edge_attr.shape[0]
    FM = W1a.shape[0]
    FM2 = W2a.shape[0]

    SP = ((S + 8 * NS - 1) // (8 * NS)) * (8 * NS)

    src2d = edge_index[0].reshape(E // 128, 128)
    nchunk = E // 128
    gpt = ((nchunk + 7) // 8 + NS - 1) // NS
    nrow_pad = NS * gpt * 8 - nchunk
    if nrow_pad:
        src2d = jnp.pad(src2d, ((0, nrow_pad), (0, 0)))
    tgt_flat = edge_index[1]

    gathered = _sc_gather(x_t, tgt_flat, E, F)
    msg = _tc_mlp1(gathered, edge_attr, W1a, b1a, W1b, b1b, E, F, FM)
    moms, cnt = _sc_moments(src2d, msg, E, SP, F, FM)
    h2 = _tc_update(moms[:, :S, :], cnt[:S], x_s, u, W2a, b2a, W2b, b2b,
                    S, F, FM, FM2)
    return _tc_norm(h2, gamma, beta, S, F)
